# baseline (speedup 1.0000x reference)
import functools

import jax
import jax.numpy as jnp
from jax import lax
from jax.experimental import pallas as pl
from jax.experimental.pallas import tpu as pltpu
from jax.experimental.pallas import tpu_sc as plsc

_NC = 2
_NS = 16
_NW = _NC * _NS
_L = 16


@jax.jit
def _gather_sc(flat_idx, table):
    n = flat_idx.shape[0]
    d = table.shape[1]
    hist = 50
    batch = n // hist
    et_n = d // 8
    bblk = batch // 128
    b_per_w = batch // _NW
    half = 256
    n_items = (b_per_w // half) * hist

    mesh = plsc.VectorSubcoreMesh(core_axis_name="c", subcore_axis_name="s")

    @functools.partial(
        pl.kernel,
        out_type=jax.ShapeDtypeStruct((hist, et_n, bblk, 8, 128), jnp.float32),
        mesh=mesh,
        scratch_types=[
            pltpu.VMEM((b_per_w * hist,), jnp.int32),
            pltpu.VMEM((2, half), jnp.int32),
            pltpu.VMEM((2, half, d), jnp.float32),
            pltpu.VMEM((2, et_n, 2, 8, 128), jnp.float32),
            pltpu.SemaphoreType.DMA,
            pltpu.SemaphoreType.DMA,
        ],
        compiler_params=pltpu.CompilerParams(
            use_tc_tiling_on_sc=False, needs_layout_passes=False
        ),
    )
    def k(idx_hbm, table_hbm, out_hbm, idx_v, ibuf, rows, tr, gsem, osem):
        wid = lax.axis_index("s") * _NC + lax.axis_index("c")
        base = wid * b_per_w * hist
        pltpu.sync_copy(idx_hbm.at[pl.ds(base, b_per_w * hist)], idx_v)

        lanes = lax.iota(jnp.int32, 16)
        lanes50 = lanes * hist

        def build_idx(i):
            h = lax.div(i, 2)
            hf = lax.rem(i, 2)
            r = lax.rem(i, 2)
            for j in range(half // _L):
                bias = (hf * half + j * _L) * hist + h
                vals = plsc.load_gather(idx_v, [lanes50 + bias])
                ibuf[r, pl.ds(j * _L, _L)] = vals

        def gather_start(i):
            r = lax.rem(i, 2)
            pltpu.async_copy(table_hbm.at[ibuf.at[r]], rows.at[r], gsem)

        def gather_wait(i):
            r = lax.rem(i, 2)
            pltpu.make_async_copy(
                table_hbm.at[ibuf.at[r]], rows.at[r], gsem
            ).wait()

        def out_descs(i, et):
            h = lax.div(i, 2)
            hf = lax.rem(i, 2)
            r = lax.rem(i, 2)
            return pltpu.make_async_copy(
                tr.at[r, et],
                out_hbm.at[h, et, pl.ds(wid * 4 + hf * 2, 2)],
                osem,
            )

        def transpose(i):
            r = lax.rem(i, 2)

            def tbody(et, carry):
                for ei in range(8):
                    e = jnp.full((16,), et * 8 + ei, jnp.int32)
                    for bb2 in range(2):
                        for j in range(128 // _L):
                            brow = lanes + (bb2 * 128 + j * _L)
                            vals = plsc.load_gather(rows.at[r], [brow, e])
                            tr[r, et, bb2, ei, pl.ds(j * _L, _L)] = vals
                return carry

            lax.fori_loop(0, et_n, tbody, 0)

        build_idx(0)
        gather_start(0)

        def body(i, carry):
            gather_wait(i)

            @pl.when(i + 1 < n_items)
            def _():
                build_idx(i + 1)
                gather_start(i + 1)

            @pl.when(i >= 2)
            def _():
                for et in range(et_n):
                    out_descs(i - 2, et).wait()

            transpose(i)
            for et in range(et_n):
                out_descs(i, et).start()
            return carry

        lax.fori_loop(0, n_items, body, 0)

        for i in (n_items - 2, n_items - 1):
            for et in range(et_n):
                out_descs(i, et).wait()

    out5 = k(flat_idx, table)
    return out5.transpose((2, 4, 0, 1, 3)).reshape(batch, hist, d)


def kernel(entity_indices, table):
    b, h = entity_indices.shape
    flat_idx = entity_indices.reshape(b * h).astype(jnp.int32)
    return _gather_sc(flat_idx, table)


# speedup vs baseline: 1.0002x; 1.0002x over previous
import functools

import jax
import jax.numpy as jnp
from jax import lax
from jax.experimental import pallas as pl
from jax.experimental.pallas import tpu as pltpu
from jax.experimental.pallas import tpu_sc as plsc

_NC = 2
_NS = 16
_NW = _NC * _NS
_L = 16


@jax.jit
def _gather_sc(flat_idx, table):
    n = flat_idx.shape[0]
    d = table.shape[1]
    hist = 50
    batch = n // hist
    et_n = d // 8
    bblk = batch // 128
    b_per_w = batch // _NW
    half = 256
    n_items = (b_per_w // half) * hist

    mesh = plsc.VectorSubcoreMesh(core_axis_name="c", subcore_axis_name="s")

    @functools.partial(
        pl.kernel,
        out_type=jax.ShapeDtypeStruct((hist, et_n, bblk, 8, 128), jnp.float32),
        mesh=mesh,
        scratch_types=[
            pltpu.VMEM((b_per_w * hist,), jnp.int32),
            pltpu.VMEM((2, half), jnp.int32),
            pltpu.VMEM((2, half, d), jnp.float32),
            pltpu.VMEM((2, et_n, 2, 8, 128), jnp.float32),
            pltpu.SemaphoreType.DMA,
            pltpu.SemaphoreType.DMA,
        ],
        compiler_params=pltpu.CompilerParams(
            use_tc_tiling_on_sc=False,
            needs_layout_passes=False,
            disable_bounds_checks=True,
        ),
    )
    def k(idx_hbm, table_hbm, out_hbm, idx_v, ibuf, rows, tr, gsem, osem):
        wid = lax.axis_index("s") * _NC + lax.axis_index("c")
        base = wid * b_per_w * hist
        pltpu.sync_copy(idx_hbm.at[pl.ds(base, b_per_w * hist)], idx_v)

        lanes = lax.iota(jnp.int32, 16)
        lanes50 = lanes * hist

        def build_idx(i):
            h = lax.div(i, 2)
            hf = lax.rem(i, 2)
            r = lax.rem(i, 2)
            for j in range(half // _L):
                bias = (hf * half + j * _L) * hist + h
                vals = plsc.load_gather(idx_v, [lanes50 + bias])
                ibuf[r, pl.ds(j * _L, _L)] = vals

        def gather_start(i):
            r = lax.rem(i, 2)
            pltpu.async_copy(table_hbm.at[ibuf.at[r]], rows.at[r], gsem)

        def gather_wait(i):
            r = lax.rem(i, 2)
            pltpu.make_async_copy(
                table_hbm.at[ibuf.at[r]], rows.at[r], gsem
            ).wait()

        def out_descs(i, et):
            h = lax.div(i, 2)
            hf = lax.rem(i, 2)
            r = lax.rem(i, 2)
            return pltpu.make_async_copy(
                tr.at[r, et],
                out_hbm.at[h, et, pl.ds(wid * 4 + hf * 2, 2)],
                osem,
            )

        def transpose(i):
            r = lax.rem(i, 2)

            def tbody(et, carry):
                for ei in range(8):
                    e = jnp.full((16,), et * 8 + ei, jnp.int32)
                    for bb2 in range(2):
                        for j in range(128 // _L):
                            brow = lanes + (bb2 * 128 + j * _L)
                            vals = plsc.load_gather(rows.at[r], [brow, e])
                            tr[r, et, bb2, ei, pl.ds(j * _L, _L)] = vals
                return carry

            lax.fori_loop(0, et_n, tbody, 0)

        build_idx(0)
        gather_start(0)

        def body(i, carry):
            gather_wait(i)

            @pl.when(i + 1 < n_items)
            def _():
                build_idx(i + 1)
                gather_start(i + 1)

            @pl.when(i >= 2)
            def _():
                for et in range(et_n):
                    out_descs(i - 2, et).wait()

            transpose(i)
            for et in range(et_n):
                out_descs(i, et).start()
            return carry

        lax.fori_loop(0, n_items, body, 0)

        for i in (n_items - 2, n_items - 1):
            for et in range(et_n):
                out_descs(i, et).wait()

    out5 = k(flat_idx, table)
    return out5.transpose((2, 4, 0, 1, 3)).reshape(batch, hist, d)


def kernel(entity_indices, table):
    b, h = entity_indices.shape
    flat_idx = entity_indices.reshape(b * h).astype(jnp.int32)
    return _gather_sc(flat_idx, table)


# speedup vs baseline: 2.4291x; 2.4287x over previous
import functools

import jax
import jax.numpy as jnp
from jax import lax
from jax.experimental import pallas as pl
from jax.experimental.pallas import tpu as pltpu
from jax.experimental.pallas import tpu_sc as plsc

_NC = 2
_NS = 16
_NW = _NC * _NS
_L = 16


@jax.jit
def _gather_sc(flat_idx, table):
    n = flat_idx.shape[0]
    d = table.shape[1]
    hist = 50
    batch = n // hist
    et_n = d // 8
    bblk = batch // 128
    b_per_w = batch // _NW
    half = 256
    n_items = (b_per_w // half) * hist

    mesh = plsc.VectorSubcoreMesh(core_axis_name="c", subcore_axis_name="s")

    @functools.partial(
        pl.kernel,
        out_type=jax.ShapeDtypeStruct((hist, et_n, bblk, 8, 128), jnp.float32),
        mesh=mesh,
        scratch_types=[
            pltpu.VMEM((b_per_w * hist,), jnp.int32),
            pltpu.VMEM((2, half), jnp.int32),
            pltpu.VMEM((2, half, d), jnp.float32),
            pltpu.VMEM((2, et_n, 2, 8, 128), jnp.float32),
            pltpu.SemaphoreType.DMA,
            pltpu.SemaphoreType.DMA,
        ],
        compiler_params=pltpu.CompilerParams(
            use_tc_tiling_on_sc=False,
            needs_layout_passes=False,
            disable_bounds_checks=True,
        ),
    )
    def k(idx_hbm, table_hbm, out_hbm, idx_v, ibuf, rows, tr, gsem, osem):
        wid = lax.axis_index("s") * _NC + lax.axis_index("c")
        base = wid * b_per_w * hist
        pltpu.sync_copy(idx_hbm.at[pl.ds(base, b_per_w * hist)], idx_v)

        lanes = lax.iota(jnp.int32, 16)
        lanes50 = lanes * hist

        def build_idx(i):
            h = lax.div(i, 2)
            hf = lax.rem(i, 2)
            r = lax.rem(i, 2)
            for j in range(half // _L):
                bias = (hf * half + j * _L) * hist + h
                vals = plsc.load_gather(idx_v, [lanes50 + bias])
                ibuf[r, pl.ds(j * _L, _L)] = vals

        def gather_start(i):
            r = lax.rem(i, 2)
            pltpu.async_copy(table_hbm.at[ibuf.at[r]], rows.at[r], gsem)

        def gather_wait(i):
            r = lax.rem(i, 2)
            pltpu.make_async_copy(
                table_hbm.at[ibuf.at[r]], rows.at[r], gsem
            ).wait()

        def out_descs(i, et):
            h = lax.div(i, 2)
            hf = lax.rem(i, 2)
            r = lax.rem(i, 2)
            return pltpu.make_async_copy(
                tr.at[r, et],
                out_hbm.at[h, et, pl.ds(wid * 4 + hf * 2, 2)],
                osem,
            )

        et_lane = lax.div(lanes, 8)
        ei_lane = lax.rem(lanes, 8)

        def transpose(i):
            r = lax.rem(i, 2)

            @functools.partial(plsc.parallel_loop, 0, half, unroll=8)
            def _(b):
                bb2 = jnp.full((_L,), lax.div(b, 128), jnp.int32)
                binv = jnp.full((_L,), lax.rem(b, 128), jnp.int32)
                for j in range(d // _L):
                    vals = rows[r, b, pl.ds(j * _L, _L)]
                    plsc.store_scatter(
                        tr.at[r], [et_lane + 2 * j, bb2, ei_lane, binv], vals
                    )

        build_idx(0)
        gather_start(0)

        def body(i, carry):
            gather_wait(i)

            @pl.when(i + 1 < n_items)
            def _():
                build_idx(i + 1)
                gather_start(i + 1)

            @pl.when(i >= 2)
            def _():
                for et in range(et_n):
                    out_descs(i - 2, et).wait()

            transpose(i)
            for et in range(et_n):
                out_descs(i, et).start()
            return carry

        lax.fori_loop(0, n_items, body, 0)

        for i in (n_items - 2, n_items - 1):
            for et in range(et_n):
                out_descs(i, et).wait()

    out5 = k(flat_idx, table)
    return out5.transpose((2, 4, 0, 1, 3)).reshape(batch, hist, d)


def kernel(entity_indices, table):
    b, h = entity_indices.shape
    flat_idx = entity_indices.reshape(b * h).astype(jnp.int32)
    return _gather_sc(flat_idx, table)
